# baseline (speedup 1.0000x reference)
import jax
import jax.numpy as jnp
from jax import lax
from jax.experimental import pallas as pl
from jax.experimental.pallas import tpu as pltpu
from jax.experimental.pallas import tpu_sc as plsc

_B = 4
_S = 8190
_SP = 8192
_NW = 32
_CW = _SP // _NW
_HALO = 128
_LD = _CW + _HALO
_VPR = _CW // 16
_DOFF = 128


def _router_body(x_hbm, pos_hbm, beh_hbm, buf, pos_buf, beh_buf, sem1, sem2):
    wid = lax.axis_index("s") * 2 + lax.axis_index("c")
    cs = pl.multiple_of(wid * _CW, _HALO)
    base = pl.multiple_of(jnp.maximum(cs - _HALO, 0), _HALO)
    off = cs - base

    pltpu.sync_copy(x_hbm.at[:, pl.ds(base, _LD)],
                    buf.at[:, pl.ds(_DOFF, _B * _LD)].reshape(_B, _LD))

    lane = lax.iota(jnp.int32, 16)
    d = (lane + 3) & 3
    is1 = d == 1
    is2 = d == 2
    d0 = d == 0
    dp1 = d + 1
    zero = jnp.zeros((16,), jnp.int32)
    one_u = jnp.ones((16,), jnp.uint32)

    def do_vec(li0, ko):
        x = buf[0, pl.ds(li0, 16)]
        x1 = buf[0, pl.ds(li0 - 1, 16)]
        x2 = buf[0, pl.ds(li0 - 2, 16)]
        x3 = buf[0, pl.ds(li0 - 3, 16)]
        v = jnp.where(is1, x1, jnp.where(is2, x2, x3))
        pos_kill = plsc.bitcast(x, jnp.uint32) <= one_u
        pos = jnp.where(pos_kill, zero, dp1)
        beh_kill = d0 | (v == 1)
        beh = jnp.where(beh_kill, zero, v)
        pos_buf[0, pl.ds(ko, 16)] = pos
        beh_buf[0, pl.ds(ko, 16)] = beh

    def row(b, _):
        li_row = _DOFF + b * _LD + off

        def body(i, _):
            k = i * 64
            for u in range(4):
                do_vec(li_row + k + u * 16, b * _CW + k + u * 16)
            return 0

        lax.fori_loop(0, _VPR // 4, body, 0)
        return 0

    lax.fori_loop(0, _B, row, 0)

    @pl.when(wid == 0)
    def _():
        l0 = lane == 0
        for b in range(_B):
            o = b * _CW
            pos_buf[0, pl.ds(o, 16)] = jnp.where(
                l0, zero, pos_buf[0, pl.ds(o, 16)])
            beh_buf[0, pl.ds(o, 16)] = jnp.where(
                l0, zero, beh_buf[0, pl.ds(o, 16)])

    @pl.when(wid == _NW - 1)
    def _():
        le = lane == 13
        for b in range(_B):
            o = b * _CW + _CW - 16
            pos_buf[0, pl.ds(o, 16)] = jnp.where(
                le, zero, pos_buf[0, pl.ds(o, 16)])

    cp1 = pltpu.make_async_copy(
        pos_buf.reshape(_B, _CW), pos_hbm.at[:, pl.ds(cs, _CW)], sem1)
    cp2 = pltpu.make_async_copy(
        beh_buf.reshape(_B, _CW), beh_hbm.at[:, pl.ds(cs, _CW)], sem2)
    cp1.start()
    cp2.start()
    cp1.wait()
    cp2.wait()


@jax.jit
def kernel(input_id_sequence):
    mesh = plsc.VectorSubcoreMesh(core_axis_name="c", subcore_axis_name="s")
    run = pl.kernel(
        _router_body,
        mesh=mesh,
        out_type=(
            jax.ShapeDtypeStruct((_B, _S), jnp.int32),
            jax.ShapeDtypeStruct((_B, _S), jnp.int32),
        ),
        scratch_types=[
            pltpu.VMEM((1, _DOFF + _B * _LD), jnp.int32),
            pltpu.VMEM((1, _B * _CW), jnp.int32),
            pltpu.VMEM((1, _B * _CW), jnp.int32),
            pltpu.SemaphoreType.DMA,
            pltpu.SemaphoreType.DMA,
        ],
    )
    return run(input_id_sequence)


# speedup vs baseline: 1.0747x; 1.0747x over previous
import jax
import jax.numpy as jnp
from jax import lax
from jax.experimental import pallas as pl
from jax.experimental.pallas import tpu as pltpu
from jax.experimental.pallas import tpu_sc as plsc

_B = 4
_S = 8190
_SP = 8192
_NW = 16
_CW = _SP // _NW
_HALO = 128
_LD = _CW + _HALO
_VPR = _CW // 16
_DOFF = 128


def _router_body(x_hbm, pos_hbm, beh_hbm, buf, pos_buf, beh_buf, sem1, sem2):
    wid = lax.axis_index("s")
    cs = pl.multiple_of(wid * _CW, _HALO)
    base = pl.multiple_of(jnp.maximum(cs - _HALO, 0), _HALO)
    off = cs - base

    pltpu.sync_copy(x_hbm.at[:, pl.ds(base, _LD)],
                    buf.at[:, pl.ds(_DOFF, _B * _LD)].reshape(_B, _LD))

    lane = lax.iota(jnp.int32, 16)
    d = (lane + 3) & 3
    is1 = d == 1
    is2 = d == 2
    d0 = d == 0
    dp1 = d + 1
    zero = jnp.zeros((16,), jnp.int32)
    one_u = jnp.ones((16,), jnp.uint32)

    def do_vec(li0, ko):
        x = buf[0, pl.ds(li0, 16)]
        x1 = buf[0, pl.ds(li0 - 1, 16)]
        x2 = buf[0, pl.ds(li0 - 2, 16)]
        x3 = buf[0, pl.ds(li0 - 3, 16)]
        v = jnp.where(is1, x1, jnp.where(is2, x2, x3))
        pos_kill = plsc.bitcast(x, jnp.uint32) <= one_u
        pos = jnp.where(pos_kill, zero, dp1)
        beh_kill = d0 | (v == 1)
        beh = jnp.where(beh_kill, zero, v)
        pos_buf[0, pl.ds(ko, 16)] = pos
        beh_buf[0, pl.ds(ko, 16)] = beh

    for b in range(_B):
        li_row = _DOFF + b * _LD + off
        ko_row = b * _CW

        @plsc.parallel_loop(0, _VPR, unroll=4)
        def _(i, _li=li_row, _ko=ko_row):
            k = i * 16
            do_vec(_li + k, _ko + k)

    @pl.when(wid == 0)
    def _():
        l0 = lane == 0
        for b in range(_B):
            o = b * _CW
            pos_buf[0, pl.ds(o, 16)] = jnp.where(
                l0, zero, pos_buf[0, pl.ds(o, 16)])
            beh_buf[0, pl.ds(o, 16)] = jnp.where(
                l0, zero, beh_buf[0, pl.ds(o, 16)])

    @pl.when(wid == _NW - 1)
    def _():
        le = lane == 13
        for b in range(_B):
            o = b * _CW + _CW - 16
            pos_buf[0, pl.ds(o, 16)] = jnp.where(
                le, zero, pos_buf[0, pl.ds(o, 16)])

    cp1 = pltpu.make_async_copy(
        pos_buf.reshape(_B, _CW), pos_hbm.at[:, pl.ds(cs, _CW)], sem1)
    cp2 = pltpu.make_async_copy(
        beh_buf.reshape(_B, _CW), beh_hbm.at[:, pl.ds(cs, _CW)], sem2)
    cp1.start()
    cp2.start()
    cp1.wait()
    cp2.wait()


@jax.jit
def kernel(input_id_sequence):
    mesh = plsc.VectorSubcoreMesh(
        core_axis_name="c", subcore_axis_name="s", num_cores=1)
    run = pl.kernel(
        _router_body,
        mesh=mesh,
        out_type=(
            jax.ShapeDtypeStruct((_B, _S), jnp.int32),
            jax.ShapeDtypeStruct((_B, _S), jnp.int32),
        ),
        scratch_types=[
            pltpu.VMEM((1, _DOFF + _B * _LD), jnp.int32),
            pltpu.VMEM((1, _B * _CW), jnp.int32),
            pltpu.VMEM((1, _B * _CW), jnp.int32),
            pltpu.SemaphoreType.DMA,
            pltpu.SemaphoreType.DMA,
        ],
    )
    return run(input_id_sequence)
